# Initial kernel scaffold; baseline (speedup 1.0000x reference)
import jax
import jax.numpy as jnp
from jax.experimental import pallas as pl


def kernel(x, weight):
    raise NotImplementedError("write your pallas kernel here")


# speedup vs baseline: 8.3055x; 8.3055x over previous
import functools

import jax
import jax.numpy as jnp
from jax import lax
from jax.experimental import pallas as pl
from jax.experimental.pallas import tpu as pltpu
from jax.experimental.pallas import tpu_sc as plsc

NUM = 100000
DIM = 128
PAD_IDX = 0

_info = plsc.get_sparse_core_info()
_NC, _NS = _info.num_cores, _info.num_subcores
_NW = _NC * _NS

_CHUNK = 256
_STREAMS = _CHUNK // 128


def _body(x_hbm, w_hbm, out_hbm, idx_v, rows_v, gsems, osems):
    wid = lax.axis_index("s") * _NC + lax.axis_index("c")
    n_total = out_hbm.shape[0]
    b_per_w = n_total // _NW
    nchunk = b_per_w // _CHUNK
    base = wid * b_per_w

    def load_idx(g, buf):
        pos = base + g * _CHUNK
        for k in range(_STREAMS):
            pltpu.sync_copy(x_hbm.at[pl.ds(pos + k * 128, 128)],
                            idx_v.at[buf, k])

    def fire_gather(g, buf):
        for k in range(_STREAMS):
            pltpu.async_copy(w_hbm.at[idx_v.at[buf, k]],
                             rows_v.at[buf, pl.ds(k * 128, 128)],
                             gsems[buf])

    def wait_gather(buf):
        for k in range(_STREAMS):
            pltpu.make_async_copy(w_hbm.at[pl.ds(0, 128)],
                                  rows_v.at[buf, pl.ds(k * 128, 128)],
                                  gsems[buf]).wait()

    def fire_out(g, buf):
        pos = base + g * _CHUNK
        pltpu.async_copy(rows_v.at[buf], out_hbm.at[pl.ds(pos, _CHUNK)],
                         osems[buf])

    def wait_out(buf):
        pltpu.make_async_copy(w_hbm.at[pl.ds(0, _CHUNK)],
                              rows_v.at[buf], osems[buf]).wait()

    def fixup(buf):
        zeros = jnp.zeros((16,), jnp.float32)
        acc = idx_v[buf, 0, pl.ds(0, 16)]
        for k in range(_STREAMS):
            for j in range(8):
                if k == 0 and j == 0:
                    continue
                acc = jnp.minimum(acc, idx_v[buf, k, pl.ds(j * 16, 16)])
        m = acc[0]
        for t in range(1, 16):
            m = jnp.minimum(m, acc[t])

        @pl.when(m == PAD_IDX)
        def _():
            def group(s, _):
                k = s // 8
                jbase = (s % 8) * 16
                idxv = idx_v[buf, k, pl.ds(jbase, 16)]
                for t in range(16):
                    @pl.when(idxv[t] == PAD_IDX)
                    def _():
                        for c in range(DIM // 16):
                            rows_v[buf, s * 16 + t, pl.ds(c * 16, 16)] = zeros
                return 0

            lax.fori_loop(0, _CHUNK // 16, group, 0)

    load_idx(0, 0)
    fire_gather(0, 0)

    def pair(p, _):
        for b in range(2):
            g = 2 * p + b
            nb = 1 - b

            @pl.when(g + 1 < nchunk)
            def _():
                @pl.when(g >= 1)
                def _():
                    wait_out(nb)
                load_idx(g + 1, nb)
                fire_gather(g + 1, nb)

            wait_gather(b)
            fixup(b)
            fire_out(g, b)
        return 0

    lax.fori_loop(0, nchunk // 2, pair, 0)
    wait_out(0)
    wait_out(1)


def kernel(x, weight):
    n = x.shape[0] * x.shape[1]
    xf = x.reshape(n).astype(jnp.int32)
    mesh = plsc.VectorSubcoreMesh(core_axis_name="c", subcore_axis_name="s")
    out = pl.kernel(
        _body,
        out_type=jax.ShapeDtypeStruct((n, DIM), jnp.float32),
        mesh=mesh,
        scratch_types=[
            pltpu.VMEM((2, _STREAMS, 128), jnp.int32),
            pltpu.VMEM((2, _CHUNK, DIM), jnp.float32),
            [pltpu.SemaphoreType.DMA, pltpu.SemaphoreType.DMA],
            [pltpu.SemaphoreType.DMA, pltpu.SemaphoreType.DMA],
        ],
    )(xf, weight)
    return out.reshape(x.shape[0], x.shape[1], DIM)
